# Initial kernel scaffold; baseline (speedup 1.0000x reference)
import jax
import jax.numpy as jnp
from jax.experimental import pallas as pl


def kernel(batch, table, pos_enc):
    raise NotImplementedError("write your pallas kernel here")


# speedup vs baseline: 4.0216x; 4.0216x over previous
import functools
import math

import jax
import jax.numpy as jnp
from jax import lax
from jax.experimental import pallas as pl
from jax.experimental.pallas import tpu as pltpu
from jax.experimental.pallas import tpu_sc as plsc

_D = 128
_SEQ = 200
_BATCH = 4096
_SCALE = math.sqrt(float(_D))
_NC, _NS, _L = 2, 16, 16
_NW = _NC * _NS
_ROWS_PER_W = _BATCH // _NW

_PREP_BLK = 2000


def _prep_body(tab_ref, out_ref):
    i = pl.program_id(0)
    rows = lax.broadcasted_iota(jnp.int32, (_PREP_BLK, 1), 0) + i * _PREP_BLK
    scale = jnp.where(rows == 0, 0.0, _SCALE)
    out_ref[...] = tab_ref[...] * scale


def _prep_table(table):
    n_rows = table.shape[0]
    grid = n_rows // _PREP_BLK
    return pl.pallas_call(
        _prep_body,
        grid=(grid,),
        in_specs=[pl.BlockSpec((_PREP_BLK, _D), lambda i: (i, 0))],
        out_specs=pl.BlockSpec((_PREP_BLK, _D), lambda i: (i, 0)),
        out_shape=jax.ShapeDtypeStruct((n_rows, _D), jnp.float32),
    )(table)


@functools.partial(
    pl.kernel,
    out_type=jax.ShapeDtypeStruct((_BATCH, _SEQ, _D), jnp.float32),
    mesh=plsc.VectorSubcoreMesh(
        core_axis_name="c", subcore_axis_name="s", num_cores=_NC, num_subcores=_NS
    ),
    scratch_types=[
        pltpu.VMEM((_SEQ, _D), jnp.float32),
        pltpu.VMEM((_SEQ,), jnp.int32),
        pltpu.VMEM((_SEQ, _D), jnp.float32),
        pltpu.SemaphoreType.DMA,
    ],
)
def _sc_gather(table_hbm, batch_hbm, pos_hbm, out_hbm, pos_v, idx_v, gbuf, sem):
    wid = lax.axis_index("s") * _NC + lax.axis_index("c")
    base = wid * _ROWS_PER_W
    pltpu.sync_copy(pos_hbm, pos_v)

    def row_body(r, carry):
        row = base + r
        pltpu.sync_copy(batch_hbm.at[row], idx_v)
        c1 = pltpu.async_copy(
            table_hbm.at[idx_v.at[pl.ds(0, 128)]], gbuf.at[pl.ds(0, 128)], sem
        )
        c2 = pltpu.async_copy(
            table_hbm.at[idx_v.at[pl.ds(128, 72)]], gbuf.at[pl.ds(128, 72)], sem
        )
        c1.wait()
        c2.wait()

        def add_body(r2, c):
            for j in range(_D // _L):
                plsc.addupdate(
                    gbuf.at[r2, pl.ds(j * _L, _L)], pos_v[r2, pl.ds(j * _L, _L)]
                )
            return c

        lax.fori_loop(0, _SEQ, add_body, None)
        pltpu.sync_copy(gbuf, out_hbm.at[row])
        return carry

    lax.fori_loop(0, _ROWS_PER_W, row_body, None)


def kernel(batch, table, pos_enc):
    table_eff = _prep_table(table)
    return _sc_gather(table_eff, batch, pos_enc)


# speedup vs baseline: 8.0279x; 1.9962x over previous
import functools
import math

import jax
import jax.numpy as jnp
from jax import lax
from jax.experimental import pallas as pl
from jax.experimental.pallas import tpu as pltpu
from jax.experimental.pallas import tpu_sc as plsc

_D = 128
_SEQ = 200
_BATCH = 4096
_SCALE = math.sqrt(float(_D))
_NC, _NS, _L = 2, 16, 16
_NW = _NC * _NS
_ROWS_PER_W = _BATCH // _NW
_NBUF = 4

_PREP_BLK = 2000


def _prep_body(tab_ref, out_ref):
    i = pl.program_id(0)
    rows = lax.broadcasted_iota(jnp.int32, (_PREP_BLK, 1), 0) + i * _PREP_BLK
    scale = jnp.where(rows == 0, 0.0, _SCALE)
    out_ref[...] = tab_ref[...] * scale


def _prep_table(table):
    n_rows = table.shape[0]
    grid = n_rows // _PREP_BLK
    return pl.pallas_call(
        _prep_body,
        grid=(grid,),
        in_specs=[pl.BlockSpec((_PREP_BLK, _D), lambda i: (i, 0))],
        out_specs=pl.BlockSpec((_PREP_BLK, _D), lambda i: (i, 0)),
        out_shape=jax.ShapeDtypeStruct((n_rows, _D), jnp.float32),
    )(table)


@functools.partial(
    pl.kernel,
    out_type=jax.ShapeDtypeStruct((_BATCH, _SEQ, _D), jnp.float32),
    mesh=plsc.VectorSubcoreMesh(
        core_axis_name="c", subcore_axis_name="s", num_cores=_NC, num_subcores=_NS
    ),
    scratch_types=[
        pltpu.VMEM((_SEQ, _D), jnp.float32),
        pltpu.VMEM((_NBUF, _SEQ), jnp.int32),
        pltpu.VMEM((_NBUF, _SEQ, _D), jnp.float32),
        pltpu.SemaphoreType.DMA((_NBUF,)),
        pltpu.SemaphoreType.DMA((_NBUF,)),
        pltpu.SemaphoreType.DMA((_NBUF,)),
    ],
)
def _sc_gather(
    table_hbm, batch_hbm, pos_hbm, out_hbm, pos_v, idx_v, gbuf, isem, gsem, ssem
):
    wid = lax.axis_index("s") * _NC + lax.axis_index("c")
    base = wid * _ROWS_PER_W
    pltpu.sync_copy(pos_hbm, pos_v)

    def fire_idx(r, b):
        pltpu.async_copy(batch_hbm.at[base + r], idx_v.at[b], isem.at[b])

    def wait_idx(b):
        pltpu.make_async_copy(batch_hbm.at[base], idx_v.at[b], isem.at[b]).wait()

    def fire_gather(b):
        pltpu.async_copy(
            table_hbm.at[idx_v.at[b, pl.ds(0, 128)]],
            gbuf.at[b, pl.ds(0, 128)],
            gsem.at[b],
        )
        pltpu.async_copy(
            table_hbm.at[idx_v.at[b, pl.ds(128, 72)]],
            gbuf.at[b, pl.ds(128, 72)],
            gsem.at[b],
        )

    def wait_gather(b):
        pltpu.make_async_copy(
            table_hbm.at[idx_v.at[b, pl.ds(0, 128)]],
            gbuf.at[b, pl.ds(0, 128)],
            gsem.at[b],
        ).wait()
        pltpu.make_async_copy(
            table_hbm.at[idx_v.at[b, pl.ds(128, 72)]],
            gbuf.at[b, pl.ds(128, 72)],
            gsem.at[b],
        ).wait()

    def add_pos(b):
        @plsc.parallel_loop(0, _SEQ, 1, unroll=2)
        def _(r2):
            for j in range(_D // _L):
                plsc.addupdate(
                    gbuf.at[b, r2, pl.ds(j * _L, _L)], pos_v[r2, pl.ds(j * _L, _L)]
                )

    def fire_scatter(r, b):
        pltpu.async_copy(gbuf.at[b], out_hbm.at[base + r], ssem.at[b])

    def wait_scatter(b):
        pltpu.make_async_copy(gbuf.at[b], out_hbm.at[base], ssem.at[b]).wait()

    for b in range(_NBUF):
        fire_idx(b, b)
    for b in range(2):
        wait_idx(b)
        fire_gather(b)

    def iteration(r, b, do_wait_scatter, do_fire_gather, do_fire_idx):
        b2 = (b + 2) % _NBUF
        wait_gather(b)
        add_pos(b)
        fire_scatter(r, b)
        if do_wait_scatter:
            wait_scatter(b2)
        if do_fire_gather:
            wait_idx(b2)
            fire_gather(b2)
        if do_fire_idx:
            fire_idx(r + 4, b)

    for b in range(_NBUF):
        iteration(b, b, b >= 2, True, True)

    @pl.loop(_NBUF, _ROWS_PER_W - _NBUF, step=_NBUF)
    def _(r0):
        for b in range(_NBUF):
            iteration(r0 + b, b, True, True, True)

    for b in range(_NBUF):
        iteration(_ROWS_PER_W - _NBUF + b, b, True, b < 2, False)

    wait_scatter(2)
    wait_scatter(3)


def kernel(batch, table, pos_enc):
    table_eff = _prep_table(table)
    return _sc_gather(table_eff, batch, pos_enc)
